# baseline (speedup 1.0000x reference)
import functools
import jax
import jax.numpy as jnp
from jax import lax
from jax.experimental import pallas as pl
from jax.experimental.pallas import tpu as pltpu
from jax.experimental.pallas import tpu_sc as plsc

N = 100000
E = 1600000
NC, NS, L = 2, 16, 16
NW = NC * NS
SPAN = 256
SHIFT = 8
NB = (N + SPAN - 1) // SPAN
NBP = 512
NP = NB * SPAN
EPW = E // NW
EPAD = 128
HCH = 8192
CH = 512
RB = 1000
GRID = N // RB

_mesh = plsc.VectorSubcoreMesh(core_axis_name="c", subcore_axis_name="s")
_sc_params = pltpu.CompilerParams(use_tc_tiling_on_sc=False)

_CHUNKS = []
_o = 0
while _o < EPW:
    _CHUNKS.append((_o, min(HCH, EPW - _o)))
    _o += HCH


def _wid():
    return lax.axis_index("s") * NC + lax.axis_index("c")


def _fill_iota(idx_v, base, count):
    def body(j, _):
        idx_v[pl.ds(j * L, L)] = (jnp.full((L,), base + j * L, jnp.int32)
                                  + lax.iota(jnp.int32, L))
        return 0
    lax.fori_loop(0, count // L, body, 0)


@functools.partial(
    pl.kernel,
    out_type=jax.ShapeDtypeStruct((NW, NBP), jnp.int32),
    mesh=_mesh,
    compiler_params=_sc_params,
    scratch_types=[
        pltpu.VMEM((HCH,), jnp.int32),
        pltpu.VMEM((HCH,), jnp.int32),
        pltpu.VMEM((NBP,), jnp.int32),
        pltpu.SemaphoreType.DMA,
    ],
)
def _hist_k(dst, cnt_out, idx_v, dst_v, cnt_v, sem):
    wid = _wid()
    ebase = wid * EPW

    def zc(j, _):
        cnt_v[pl.ds(j * L, L)] = jnp.zeros((L,), jnp.int32)
        return 0
    lax.fori_loop(0, NBP // L, zc, 0)

    for (coff, clen) in _CHUNKS:
        _fill_iota(idx_v, ebase + coff, clen)
        pltpu.async_copy(dst.at[idx_v.at[pl.ds(0, clen)]],
                         dst_v.at[pl.ds(0, clen)], sem).wait()

        def hist(i, _):
            d = dst_v[pl.ds(i, 1)][0]
            b = d >> SHIFT
            cnt_v[pl.ds(b, 1)] = cnt_v[pl.ds(b, 1)] + 1
            return 0
        lax.fori_loop(0, clen, hist, 0)

    pltpu.sync_copy(cnt_v, cnt_out.at[wid])


def _off_body(cnt_ref, off_ref, bptr_ref):
    ci = cnt_ref[...]
    tot = jnp.sum(ci, axis=0, keepdims=True)
    s = tot
    k = 1
    while k < NBP:
        s = s + jnp.concatenate(
            [jnp.zeros((1, k), jnp.int32), s[:, :-k]], axis=1)
        k *= 2
    bptr = s - tot
    wp = ci
    k = 1
    while k < NW:
        wp = wp + jnp.concatenate(
            [jnp.zeros((k, NBP), jnp.int32), wp[:-k, :]], axis=0)
        k *= 2
    off_ref[...] = bptr + (wp - ci)
    bptr_ref[...] = bptr


def _offsets(counts):
    return pl.pallas_call(
        _off_body,
        out_shape=[jax.ShapeDtypeStruct((NW, NBP), jnp.int32),
                   jax.ShapeDtypeStruct((1, NBP), jnp.int32)],
    )(counts)


@functools.partial(
    pl.kernel,
    out_type=[jax.ShapeDtypeStruct((E + EPAD,), jnp.int32),
              jax.ShapeDtypeStruct((E + EPAD,), jnp.int32)],
    mesh=_mesh,
    compiler_params=_sc_params,
    scratch_types=[
        pltpu.VMEM((HCH,), jnp.int32),
        pltpu.VMEM((HCH,), jnp.int32),
        pltpu.VMEM((HCH,), jnp.int32),
        pltpu.VMEM((CH,), jnp.int32),
        pltpu.VMEM((NBP,), jnp.int32),
        pltpu.VMEM((NBP,), jnp.int32),
        pltpu.SemaphoreType.DMA,
        pltpu.SemaphoreType.DMA,
    ],
)
def _place_k(src, dst, off, out_src, out_dst, idx_v, src_v, dst_v, pos_v,
             cnt_v, off_v, sem1, sem2):
    wid = _wid()
    ebase = wid * EPW

    def zc(j, _):
        cnt_v[pl.ds(j * L, L)] = jnp.zeros((L,), jnp.int32)
        return 0
    lax.fori_loop(0, NBP // L, zc, 0)
    pltpu.sync_copy(off.at[wid], off_v)

    @pl.when(wid == 0)
    def _():
        _fill_iota(idx_v, E, EPAD)
        pltpu.async_copy(cnt_v.at[pl.ds(0, EPAD)],
                         out_src.at[idx_v.at[pl.ds(0, EPAD)]], sem1).wait()
        pltpu.async_copy(cnt_v.at[pl.ds(0, EPAD)],
                         out_dst.at[idx_v.at[pl.ds(0, EPAD)]], sem2).wait()

    for (coff, clen) in _CHUNKS:
        _fill_iota(idx_v, ebase + coff, clen)
        cpa = pltpu.async_copy(src.at[idx_v.at[pl.ds(0, clen)]],
                               src_v.at[pl.ds(0, clen)], sem1)
        cpb = pltpu.async_copy(dst.at[idx_v.at[pl.ds(0, clen)]],
                               dst_v.at[pl.ds(0, clen)], sem2)
        cpa.wait()
        cpb.wait()

        def place_chunk(base, size):
            def place(i, _):
                d = dst_v[pl.ds(base + i, 1)][0]
                b = d >> SHIFT
                cur = cnt_v[pl.ds(b, 1)][0]
                pos_v[pl.ds(i, 1)] = off_v[pl.ds(b, 1)] + cur
                cnt_v[pl.ds(b, 1)] = cnt_v[pl.ds(b, 1)] + 1
                return 0
            lax.fori_loop(0, size, place, 0)
            cp1 = pltpu.async_copy(src_v.at[pl.ds(base, size)],
                                   out_src.at[pos_v.at[pl.ds(0, size)]], sem1)
            cp2 = pltpu.async_copy(dst_v.at[pl.ds(base, size)],
                                   out_dst.at[pos_v.at[pl.ds(0, size)]], sem2)
            cp1.wait()
            cp2.wait()

        n_full = clen // CH
        tail = clen % CH

        def sub(k, _):
            place_chunk(k * CH, CH)
            return 0
        lax.fori_loop(0, n_full, sub, 0)
        if tail:
            place_chunk(n_full * CH, tail)


def _make_edge_kernel(D, H, F, C):
    nvr = D // L

    @functools.partial(
        pl.kernel,
        out_type=jax.ShapeDtypeStruct((NP, D), jnp.float32),
        mesh=_mesh,
        compiler_params=_sc_params,
        scratch_types=[
            pltpu.VMEM((NBP,), jnp.int32),
            pltpu.VMEM((L,), jnp.float32),
            pltpu.VMEM((C,), jnp.int32),
            pltpu.VMEM((C,), jnp.int32),
            pltpu.VMEM((C,), jnp.int32),
            pltpu.VMEM((C, D), jnp.float32),
            pltpu.VMEM((C, L), jnp.float32),
            pltpu.VMEM((C, L), jnp.float32),
            pltpu.VMEM((SPAN, D), jnp.float32),
            pltpu.VMEM((SPAN, L), jnp.float32),
            pltpu.SemaphoreType.DMA,
            pltpu.SemaphoreType.DMA,
            pltpu.SemaphoreType.DMA,
        ],
    )
    def k(hfeat, asrc, adst, ssrc, sdst, bptr, cvec, out,
          bptr_v, cv, eidx, ssrc_v, sdst_v, rows, asv, adv, acc, den,
          s1, s2, s3):
        wid = _wid()
        pltpu.sync_copy(bptr.at[0], bptr_v)
        pltpu.sync_copy(cvec.at[0], cv)
        cvv = cv[...]
        nown = (NB - wid + NW - 1) // NW

        def bucket_body(kk, _):
            b = wid + kk * NW
            nbase = b * SPAN

            def zrow(r, _):
                for j in range(nvr):
                    acc[r, pl.ds(j * L, L)] = jnp.zeros((L,), jnp.float32)
                den[r, :] = jnp.zeros((L,), jnp.float32)
                return 0
            lax.fori_loop(0, SPAN, zrow, 0)

            e0 = bptr_v[pl.ds(b, 1)][0]
            e1 = bptr_v[pl.ds(b + 1, 1)][0]
            ec = e1 - e0
            nch = (ec + C - 1) // C

            def chunk_body(ch, _):
                ebase2 = e0 + ch * C
                _fill_iota(eidx, ebase2, C)
                cpa = pltpu.async_copy(ssrc.at[eidx], ssrc_v, s1)
                cpb = pltpu.async_copy(sdst.at[eidx], sdst_v, s2)
                cpa.wait()
                cpb.wait()
                cp1 = pltpu.async_copy(hfeat.at[ssrc_v], rows, s1)
                cp2 = pltpu.async_copy(asrc.at[ssrc_v], asv, s2)
                cp3 = pltpu.async_copy(adst.at[sdst_v], adv, s3)
                cp1.wait()
                cp2.wait()
                cp3.wait()
                rem = jnp.minimum(C, ec - ch * C)

                def wpass(e, _):
                    ev = asv[e, :] + adv[e, :]
                    ev = jnp.maximum(ev, 0.2 * ev)
                    asv[e, :] = jnp.exp(ev - cvv)
                    return 0
                lax.fori_loop(0, rem, wpass, 0)

                def do_edge(e):
                    r = sdst_v[pl.ds(e, 1)][0] - nbase
                    wv = asv[e, :]
                    plsc.addupdate(den.at[r], wv)
                    for h in range(H):
                        bc = jnp.full((L,), wv[h])
                        for j in range(F // L):
                            o = h * F + j * L
                            plsc.addupdate(acc.at[r, pl.ds(o, L)],
                                           bc * rows[e, pl.ds(o, L)])

                def edge2(g, _):
                    do_edge(g * 2)
                    do_edge(g * 2 + 1)
                    return 0
                lax.fori_loop(0, rem // 2, edge2, 0)

                def edge1(e, _):
                    do_edge(e)
                    return 0
                lax.fori_loop((rem // 2) * 2, rem, edge1, 0)
                return 0
            lax.fori_loop(0, nch, chunk_body, 0)

            def flush_body(r, _):
                drow = den[r, :]
                for h in range(H):
                    dv = jnp.full((L,), drow[h]) + 1e-16
                    inv = 1.0 / dv
                    for j in range(F // L):
                        o = h * F + j * L
                        acc[r, pl.ds(o, L)] = acc[r, pl.ds(o, L)] * inv
                return 0
            lax.fori_loop(0, SPAN, flush_body, 0)
            pltpu.sync_copy(acc, out.at[pl.ds(nbase, SPAN)])
            return 0
        lax.fori_loop(0, nown, bucket_body, 0)

    return k


_edge_k1 = _make_edge_kernel(256, 4, 64, 64)
_edge_k2 = _make_edge_kernel(384, 4, 96, 48)
_edge_k3 = _make_edge_kernel(48, 1, 48, 128)


def _prep_body(x_ref, w_ref, asm_ref, adm_ref, h_ref, as_ref, ad_ref, c_ref,
               ms_ref, md_ref):
    i = pl.program_id(0)
    h = jnp.dot(x_ref[...], w_ref[...], preferred_element_type=jnp.float32)
    h_ref[...] = h
    a_s = jnp.dot(h, asm_ref[...], preferred_element_type=jnp.float32)
    a_d = jnp.dot(h, adm_ref[...], preferred_element_type=jnp.float32)
    as_ref[...] = a_s
    ad_ref[...] = a_d
    bs = jnp.max(a_s, axis=0, keepdims=True)
    bd = jnp.max(a_d, axis=0, keepdims=True)

    @pl.when(i == 0)
    def _():
        ms_ref[...] = bs
        md_ref[...] = bd

    @pl.when(i > 0)
    def _():
        ms_ref[...] = jnp.maximum(ms_ref[...], bs)
        md_ref[...] = jnp.maximum(md_ref[...], bd)

    m = ms_ref[...] + md_ref[...]
    c_ref[...] = jnp.maximum(m, 0.2 * m)


def _prep(x, w, asm, adm, K, DOUT):
    return pl.pallas_call(
        _prep_body,
        grid=(GRID,),
        in_specs=[
            pl.BlockSpec((RB, K), lambda i: (i, 0)),
            pl.BlockSpec((K, DOUT), lambda i: (0, 0)),
            pl.BlockSpec((DOUT, 16), lambda i: (0, 0)),
            pl.BlockSpec((DOUT, 16), lambda i: (0, 0)),
        ],
        out_specs=[
            pl.BlockSpec((RB, DOUT), lambda i: (i, 0)),
            pl.BlockSpec((RB, 16), lambda i: (i, 0)),
            pl.BlockSpec((RB, 16), lambda i: (i, 0)),
            pl.BlockSpec((1, 16), lambda i: (0, 0)),
        ],
        out_shape=[
            jax.ShapeDtypeStruct((N, DOUT), jnp.float32),
            jax.ShapeDtypeStruct((N, 16), jnp.float32),
            jax.ShapeDtypeStruct((N, 16), jnp.float32),
            jax.ShapeDtypeStruct((1, 16), jnp.float32),
        ],
        scratch_shapes=[pltpu.VMEM((1, 16), jnp.float32),
                        pltpu.VMEM((1, 16), jnp.float32)],
    )(x, w, asm, adm)


def _stats_body(z_ref, b_ref, sum_ref, sq_ref):
    i = pl.program_id(0)
    t = z_ref[...] + b_ref[...]
    s1 = jnp.sum(t, axis=0, keepdims=True)
    s2 = jnp.sum(t * t, axis=0, keepdims=True)

    @pl.when(i == 0)
    def _():
        sum_ref[...] = s1
        sq_ref[...] = s2

    @pl.when(i > 0)
    def _():
        sum_ref[...] = sum_ref[...] + s1
        sq_ref[...] = sq_ref[...] + s2


def _stats(z, b, D):
    return pl.pallas_call(
        _stats_body,
        grid=(GRID,),
        in_specs=[
            pl.BlockSpec((RB, D), lambda i: (i, 0)),
            pl.BlockSpec((1, D), lambda i: (0, 0)),
        ],
        out_specs=[
            pl.BlockSpec((1, D), lambda i: (0, 0)),
            pl.BlockSpec((1, D), lambda i: (0, 0)),
        ],
        out_shape=[jax.ShapeDtypeStruct((1, D), jnp.float32),
                   jax.ShapeDtypeStruct((1, D), jnp.float32)],
    )(z, b)


def _apply_prep_body(z_ref, b_ref, gw_ref, gb_ref, gms_ref, s1_ref, s2_ref,
                     w_ref, asm_ref, adm_ref, h_ref, as_ref, ad_ref, c_ref,
                     ms_ref, md_ref):
    i = pl.program_id(0)
    t = z_ref[...] + b_ref[...]
    m = s1_ref[...] * (1.0 / N)
    mm = m * gms_ref[...]
    var = s2_ref[...] * (1.0 / N) - 2.0 * mm * m + mm * mm
    y = gw_ref[...] * (t - mm) / jnp.sqrt(var + 1e-5) + gb_ref[...]
    a = jnp.where(y > 0, y, jnp.exp(y) - 1.0)
    h = jnp.dot(a, w_ref[...], preferred_element_type=jnp.float32)
    h_ref[...] = h
    a_s = jnp.dot(h, asm_ref[...], preferred_element_type=jnp.float32)
    a_d = jnp.dot(h, adm_ref[...], preferred_element_type=jnp.float32)
    as_ref[...] = a_s
    ad_ref[...] = a_d
    bs = jnp.max(a_s, axis=0, keepdims=True)
    bd = jnp.max(a_d, axis=0, keepdims=True)

    @pl.when(i == 0)
    def _():
        ms_ref[...] = bs
        md_ref[...] = bd

    @pl.when(i > 0)
    def _():
        ms_ref[...] = jnp.maximum(ms_ref[...], bs)
        md_ref[...] = jnp.maximum(md_ref[...], bd)

    mx = ms_ref[...] + md_ref[...]
    c_ref[...] = jnp.maximum(mx, 0.2 * mx)


def _apply_prep(z, b, gw, gb, gms, s1, s2, w, asm, adm, D, DOUT):
    return pl.pallas_call(
        _apply_prep_body,
        grid=(GRID,),
        in_specs=[
            pl.BlockSpec((RB, D), lambda i: (i, 0)),
            pl.BlockSpec((1, D), lambda i: (0, 0)),
            pl.BlockSpec((1, D), lambda i: (0, 0)),
            pl.BlockSpec((1, D), lambda i: (0, 0)),
            pl.BlockSpec((1, D), lambda i: (0, 0)),
            pl.BlockSpec((1, D), lambda i: (0, 0)),
            pl.BlockSpec((1, D), lambda i: (0, 0)),
            pl.BlockSpec((D, DOUT), lambda i: (0, 0)),
            pl.BlockSpec((DOUT, 16), lambda i: (0, 0)),
            pl.BlockSpec((DOUT, 16), lambda i: (0, 0)),
        ],
        out_specs=[
            pl.BlockSpec((RB, DOUT), lambda i: (i, 0)),
            pl.BlockSpec((RB, 16), lambda i: (i, 0)),
            pl.BlockSpec((RB, 16), lambda i: (i, 0)),
            pl.BlockSpec((1, 16), lambda i: (0, 0)),
        ],
        out_shape=[
            jax.ShapeDtypeStruct((N, DOUT), jnp.float32),
            jax.ShapeDtypeStruct((N, 16), jnp.float32),
            jax.ShapeDtypeStruct((N, 16), jnp.float32),
            jax.ShapeDtypeStruct((1, 16), jnp.float32),
        ],
        scratch_shapes=[pltpu.VMEM((1, 16), jnp.float32),
                        pltpu.VMEM((1, 16), jnp.float32)],
    )(z, b, gw, gb, gms, s1, s2, w, asm, adm)


def _final_body(z_ref, b_ref, o_ref):
    o_ref[...] = z_ref[...] + b_ref[...]


def _final(z, b, D):
    return pl.pallas_call(
        _final_body,
        grid=(GRID,),
        in_specs=[
            pl.BlockSpec((RB, D), lambda i: (i, 0)),
            pl.BlockSpec((1, D), lambda i: (0, 0)),
        ],
        out_specs=pl.BlockSpec((RB, D), lambda i: (i, 0)),
        out_shape=jax.ShapeDtypeStruct((N, D), jnp.float32),
    )(z, b)


def _amap(a, H, F):
    m = jnp.zeros((H * F, 16), jnp.float32)
    for h in range(H):
        m = m.at[h * F:(h + 1) * F, h].set(a[h])
    return m


def kernel(x, edge_index, W1, a1s, a1d, b1, gn1_w, gn1_b, gn1_ms,
           W2, a2s, a2d, b2, gn2_w, gn2_b, gn2_ms, W3, a3s, a3d, b3):
    src = edge_index[0]
    dst = edge_index[1]

    counts = _hist_k(dst)
    off, bptr = _offsets(counts)
    ssrc, sdst = _place_k(src, dst, off)

    xp = jnp.pad(x, ((0, 0), (0, 128 - 11)))
    w1p = jnp.pad(W1, ((0, 128 - 11), (0, 0)))
    h1, as1, ad1, c1 = _prep(xp, w1p, _amap(a1s, 4, 64), _amap(a1d, 4, 64),
                             128, 256)
    z1 = _edge_k1(h1, as1, ad1, ssrc, sdst, bptr, c1)

    s1a, s1b = _stats(z1, b1[None, :], 256)
    h2, as2, ad2, c2 = _apply_prep(z1, b1[None, :], gn1_w[None, :],
                                   gn1_b[None, :], gn1_ms[None, :], s1a, s1b,
                                   W2, _amap(a2s, 4, 96), _amap(a2d, 4, 96),
                                   256, 384)
    z2 = _edge_k2(h2, as2, ad2, ssrc, sdst, bptr, c2)

    s2a, s2b = _stats(z2, b2[None, :], 384)
    h3, as3, ad3, c3 = _apply_prep(z2, b2[None, :], gn2_w[None, :],
                                   gn2_b[None, :], gn2_ms[None, :], s2a, s2b,
                                   W3, _amap(a3s, 1, 48), _amap(a3d, 1, 48),
                                   384, 48)
    z3 = _edge_k3(h3, as3, ad3, ssrc, sdst, bptr, c3)

    return _final(z3, b3[None, :], 48)


# speedup vs baseline: 1.2732x; 1.2732x over previous
import functools
import jax
import jax.numpy as jnp
from jax import lax
from jax.experimental import pallas as pl
from jax.experimental.pallas import tpu as pltpu
from jax.experimental.pallas import tpu_sc as plsc

N = 100000
E = 1600000
NC, NS, L = 2, 16, 16
NW = NC * NS
SPAN = 128
SHIFT = 7
NB = (N + SPAN - 1) // SPAN
NBP = 1024
NP = NB * SPAN
EPW = E // NW
EPAD = 128
HCH = 8192
CH = 512
RB = 1000
GRID = N // RB

_mesh = plsc.VectorSubcoreMesh(core_axis_name="c", subcore_axis_name="s")
_sc_params = pltpu.CompilerParams(use_tc_tiling_on_sc=False)

_CHUNKS = []
_o = 0
while _o < EPW:
    _CHUNKS.append((_o, min(HCH, EPW - _o)))
    _o += HCH


def _wid():
    return lax.axis_index("s") * NC + lax.axis_index("c")


def _fill_iota(idx_v, base, count):
    def body(j, _):
        idx_v[pl.ds(j * L, L)] = (jnp.full((L,), base + j * L, jnp.int32)
                                  + lax.iota(jnp.int32, L))
        return 0
    lax.fori_loop(0, count // L, body, 0)


@functools.partial(
    pl.kernel,
    out_type=jax.ShapeDtypeStruct((NW, NBP), jnp.int32),
    mesh=_mesh,
    compiler_params=_sc_params,
    scratch_types=[
        pltpu.VMEM((HCH,), jnp.int32),
        pltpu.VMEM((HCH,), jnp.int32),
        pltpu.VMEM((NBP,), jnp.int32),
        pltpu.SemaphoreType.DMA,
    ],
)
def _hist_k(dst, cnt_out, idx_v, dst_v, cnt_v, sem):
    wid = _wid()
    ebase = wid * EPW

    def zc(j, _):
        cnt_v[pl.ds(j * L, L)] = jnp.zeros((L,), jnp.int32)
        return 0
    lax.fori_loop(0, NBP // L, zc, 0)

    for (coff, clen) in _CHUNKS:
        _fill_iota(idx_v, ebase + coff, clen)
        pltpu.async_copy(dst.at[idx_v.at[pl.ds(0, clen)]],
                         dst_v.at[pl.ds(0, clen)], sem).wait()

        def hist(i, _):
            d = dst_v[pl.ds(i, 1)][0]
            b = d >> SHIFT
            cnt_v[pl.ds(b, 1)] = cnt_v[pl.ds(b, 1)] + 1
            return 0
        lax.fori_loop(0, clen, hist, 0)

    pltpu.sync_copy(cnt_v, cnt_out.at[wid])


def _off_body(cnt_ref, off_ref, bptr_ref):
    ci = cnt_ref[...]
    tot = jnp.sum(ci, axis=0, keepdims=True)
    s = tot
    k = 1
    while k < NBP:
        s = s + jnp.concatenate(
            [jnp.zeros((1, k), jnp.int32), s[:, :-k]], axis=1)
        k *= 2
    bptr = s - tot
    wp = ci
    k = 1
    while k < NW:
        wp = wp + jnp.concatenate(
            [jnp.zeros((k, NBP), jnp.int32), wp[:-k, :]], axis=0)
        k *= 2
    off_ref[...] = bptr + (wp - ci)
    bptr_ref[...] = bptr


def _offsets(counts):
    return pl.pallas_call(
        _off_body,
        out_shape=[jax.ShapeDtypeStruct((NW, NBP), jnp.int32),
                   jax.ShapeDtypeStruct((1, NBP), jnp.int32)],
    )(counts)


@functools.partial(
    pl.kernel,
    out_type=[jax.ShapeDtypeStruct((E + EPAD,), jnp.int32),
              jax.ShapeDtypeStruct((E + EPAD,), jnp.int32)],
    mesh=_mesh,
    compiler_params=_sc_params,
    scratch_types=[
        pltpu.VMEM((HCH,), jnp.int32),
        pltpu.VMEM((HCH,), jnp.int32),
        pltpu.VMEM((HCH,), jnp.int32),
        pltpu.VMEM((CH,), jnp.int32),
        pltpu.VMEM((NBP,), jnp.int32),
        pltpu.VMEM((NBP,), jnp.int32),
        pltpu.SemaphoreType.DMA,
        pltpu.SemaphoreType.DMA,
    ],
)
def _place_k(src, dst, off, out_src, out_dst, idx_v, src_v, dst_v, pos_v,
             cnt_v, off_v, sem1, sem2):
    wid = _wid()
    ebase = wid * EPW

    def zc(j, _):
        cnt_v[pl.ds(j * L, L)] = jnp.zeros((L,), jnp.int32)
        return 0
    lax.fori_loop(0, NBP // L, zc, 0)
    pltpu.sync_copy(off.at[wid], off_v)

    @pl.when(wid == 0)
    def _():
        _fill_iota(idx_v, E, EPAD)
        pltpu.async_copy(cnt_v.at[pl.ds(0, EPAD)],
                         out_src.at[idx_v.at[pl.ds(0, EPAD)]], sem1).wait()
        pltpu.async_copy(cnt_v.at[pl.ds(0, EPAD)],
                         out_dst.at[idx_v.at[pl.ds(0, EPAD)]], sem2).wait()

    for (coff, clen) in _CHUNKS:
        _fill_iota(idx_v, ebase + coff, clen)
        cpa = pltpu.async_copy(src.at[idx_v.at[pl.ds(0, clen)]],
                               src_v.at[pl.ds(0, clen)], sem1)
        cpb = pltpu.async_copy(dst.at[idx_v.at[pl.ds(0, clen)]],
                               dst_v.at[pl.ds(0, clen)], sem2)
        cpa.wait()
        cpb.wait()

        def place_chunk(base, size):
            def place(i, _):
                d = dst_v[pl.ds(base + i, 1)][0]
                b = d >> SHIFT
                cur = cnt_v[pl.ds(b, 1)][0]
                pos_v[pl.ds(i, 1)] = off_v[pl.ds(b, 1)] + cur
                cnt_v[pl.ds(b, 1)] = cnt_v[pl.ds(b, 1)] + 1
                return 0
            lax.fori_loop(0, size, place, 0)
            cp1 = pltpu.async_copy(src_v.at[pl.ds(base, size)],
                                   out_src.at[pos_v.at[pl.ds(0, size)]], sem1)
            cp2 = pltpu.async_copy(dst_v.at[pl.ds(base, size)],
                                   out_dst.at[pos_v.at[pl.ds(0, size)]], sem2)
            cp1.wait()
            cp2.wait()

        n_full = clen // CH
        tail = clen % CH

        def sub(k, _):
            place_chunk(k * CH, CH)
            return 0
        lax.fori_loop(0, n_full, sub, 0)
        if tail:
            place_chunk(n_full * CH, tail)


def _make_edge_kernel(D, H, F, C):
    nvr = D // L

    @functools.partial(
        pl.kernel,
        out_type=jax.ShapeDtypeStruct((NP, D), jnp.float32),
        mesh=_mesh,
        compiler_params=_sc_params,
        scratch_types=[
            pltpu.VMEM((NBP,), jnp.int32),
            pltpu.VMEM((L,), jnp.float32),
            pltpu.VMEM((C,), jnp.int32),
            pltpu.VMEM((C,), jnp.int32),
            pltpu.VMEM((C,), jnp.int32),
            pltpu.VMEM((C,), jnp.int32),
            pltpu.VMEM((C,), jnp.int32),
            pltpu.VMEM((C,), jnp.int32),
            pltpu.VMEM((C, D), jnp.float32),
            pltpu.VMEM((C, D), jnp.float32),
            pltpu.VMEM((C, L), jnp.float32),
            pltpu.VMEM((C, L), jnp.float32),
            pltpu.VMEM((C, L), jnp.float32),
            pltpu.VMEM((C, L), jnp.float32),
            pltpu.VMEM((SPAN, D), jnp.float32),
            pltpu.VMEM((SPAN, L), jnp.float32),
            pltpu.SemaphoreType.DMA,
            pltpu.SemaphoreType.DMA,
            pltpu.SemaphoreType.DMA,
            pltpu.SemaphoreType.DMA,
            pltpu.SemaphoreType.DMA,
            pltpu.SemaphoreType.DMA,
        ],
    )
    def k(hfeat, asrc, adst, ssrc, sdst, bptr, cvec, out,
          bptr_v, cv, eidx0, eidx1, ssrc0, ssrc1, sdst0, sdst1,
          rows0, rows1, asv0, asv1, adv0, adv1, acc, den,
          si0, si1, sr0, sr1, sa0, sa1):
        eidx = [eidx0, eidx1]
        ssrc_v = [ssrc0, ssrc1]
        sdst_v = [sdst0, sdst1]
        rows = [rows0, rows1]
        asv = [asv0, asv1]
        adv = [adv0, adv1]
        si = [si0, si1]
        sr = [sr0, sr1]
        sa = [sa0, sa1]

        wid = _wid()
        pltpu.sync_copy(bptr.at[0], bptr_v)
        pltpu.sync_copy(cvec.at[0], cv)
        cvv = cv[...]
        nown = (NB - wid + NW - 1) // NW

        def issue_idx(x, e0, ch):
            _fill_iota(eidx[x], e0 + ch * C, C)
            pltpu.async_copy(ssrc.at[eidx[x]], ssrc_v[x], si[x])
            pltpu.async_copy(sdst.at[eidx[x]], sdst_v[x], si[x])

        def wait_idx(x):
            pltpu.make_async_copy(ssrc.at[eidx[x]], ssrc_v[x], si[x]).wait()
            pltpu.make_async_copy(sdst.at[eidx[x]], sdst_v[x], si[x]).wait()

        def issue_rows(x):
            pltpu.async_copy(hfeat.at[ssrc_v[x]], rows[x], sr[x])
            pltpu.async_copy(asrc.at[ssrc_v[x]], asv[x], sa[x])
            pltpu.async_copy(adst.at[sdst_v[x]], adv[x], sa[x])

        def wait_rows(x):
            pltpu.make_async_copy(hfeat.at[ssrc_v[x]], rows[x], sr[x]).wait()
            pltpu.make_async_copy(asrc.at[ssrc_v[x]], asv[x], sa[x]).wait()
            pltpu.make_async_copy(adst.at[sdst_v[x]], adv[x], sa[x]).wait()

        def bucket_body(kk, _):
            b = wid + kk * NW
            nbase = b * SPAN

            def zrow(r, _):
                for j in range(nvr):
                    acc[r, pl.ds(j * L, L)] = jnp.zeros((L,), jnp.float32)
                den[r, :] = jnp.zeros((L,), jnp.float32)
                return 0
            lax.fori_loop(0, SPAN, zrow, 0)

            e0 = bptr_v[pl.ds(b, 1)][0]
            e1 = bptr_v[pl.ds(b + 1, 1)][0]
            ec = e1 - e0
            nch = (ec + C - 1) // C

            @pl.when(nch > 0)
            def _():
                issue_idx(0, e0, 0)
                wait_idx(0)
                issue_rows(0)

                @pl.when(nch > 1)
                def _():
                    issue_idx(1, e0, 1)

            def process(x, ch):
                wait_rows(x)

                @pl.when(ch + 1 < nch)
                def _():
                    wait_idx(1 - x)
                    issue_rows(1 - x)

                rem = jnp.minimum(C, ec - ch * C)
                rws = rows[x]
                sdv = sdst_v[x]
                asx = asv[x]
                adx = adv[x]

                def edge_body(e, _):
                    r = sdv[pl.ds(e, 1)][0] - nbase
                    ev = asx[e, :] + adx[e, :]
                    ev = jnp.maximum(ev, 0.2 * ev)
                    wv = jnp.exp(ev - cvv)
                    plsc.addupdate(den.at[r], wv)
                    for h in range(H):
                        bc = jnp.full((L,), wv[h])
                        for j in range(F // L):
                            o = h * F + j * L
                            plsc.addupdate(acc.at[r, pl.ds(o, L)],
                                           bc * rws[e, pl.ds(o, L)])
                    return 0
                lax.fori_loop(0, rem, edge_body, 0)

                @pl.when(ch + 2 < nch)
                def _():
                    issue_idx(x, e0, ch + 2)

            def chunk_body(ch, _):
                @pl.when(ch % 2 == 0)
                def _():
                    process(0, ch)

                @pl.when(ch % 2 == 1)
                def _():
                    process(1, ch)
                return 0
            lax.fori_loop(0, nch, chunk_body, 0)

            def flush_body(r, _):
                drow = den[r, :]
                for h in range(H):
                    dv = jnp.full((L,), drow[h]) + 1e-16
                    inv = 1.0 / dv
                    for j in range(F // L):
                        o = h * F + j * L
                        acc[r, pl.ds(o, L)] = acc[r, pl.ds(o, L)] * inv
                return 0
            lax.fori_loop(0, SPAN, flush_body, 0)
            pltpu.sync_copy(acc, out.at[pl.ds(nbase, SPAN)])
            return 0
        lax.fori_loop(0, nown, bucket_body, 0)

    return k


_edge_k1 = _make_edge_kernel(256, 4, 64, 64)
_edge_k2 = _make_edge_kernel(384, 4, 96, 64)
_edge_k3 = _make_edge_kernel(48, 1, 48, 128)


def _prep_body(x_ref, w_ref, asm_ref, adm_ref, h_ref, as_ref, ad_ref, c_ref,
               ms_ref, md_ref):
    i = pl.program_id(0)
    h = jnp.dot(x_ref[...], w_ref[...], preferred_element_type=jnp.float32)
    h_ref[...] = h
    a_s = jnp.dot(h, asm_ref[...], preferred_element_type=jnp.float32)
    a_d = jnp.dot(h, adm_ref[...], preferred_element_type=jnp.float32)
    as_ref[...] = a_s
    ad_ref[...] = a_d
    bs = jnp.max(a_s, axis=0, keepdims=True)
    bd = jnp.max(a_d, axis=0, keepdims=True)

    @pl.when(i == 0)
    def _():
        ms_ref[...] = bs
        md_ref[...] = bd

    @pl.when(i > 0)
    def _():
        ms_ref[...] = jnp.maximum(ms_ref[...], bs)
        md_ref[...] = jnp.maximum(md_ref[...], bd)

    m = ms_ref[...] + md_ref[...]
    c_ref[...] = jnp.maximum(m, 0.2 * m)


def _prep(x, w, asm, adm, K, DOUT):
    return pl.pallas_call(
        _prep_body,
        grid=(GRID,),
        in_specs=[
            pl.BlockSpec((RB, K), lambda i: (i, 0)),
            pl.BlockSpec((K, DOUT), lambda i: (0, 0)),
            pl.BlockSpec((DOUT, 16), lambda i: (0, 0)),
            pl.BlockSpec((DOUT, 16), lambda i: (0, 0)),
        ],
        out_specs=[
            pl.BlockSpec((RB, DOUT), lambda i: (i, 0)),
            pl.BlockSpec((RB, 16), lambda i: (i, 0)),
            pl.BlockSpec((RB, 16), lambda i: (i, 0)),
            pl.BlockSpec((1, 16), lambda i: (0, 0)),
        ],
        out_shape=[
            jax.ShapeDtypeStruct((N, DOUT), jnp.float32),
            jax.ShapeDtypeStruct((N, 16), jnp.float32),
            jax.ShapeDtypeStruct((N, 16), jnp.float32),
            jax.ShapeDtypeStruct((1, 16), jnp.float32),
        ],
        scratch_shapes=[pltpu.VMEM((1, 16), jnp.float32),
                        pltpu.VMEM((1, 16), jnp.float32)],
    )(x, w, asm, adm)


def _stats_body(z_ref, b_ref, sum_ref, sq_ref):
    i = pl.program_id(0)
    t = z_ref[...] + b_ref[...]
    s1 = jnp.sum(t, axis=0, keepdims=True)
    s2 = jnp.sum(t * t, axis=0, keepdims=True)

    @pl.when(i == 0)
    def _():
        sum_ref[...] = s1
        sq_ref[...] = s2

    @pl.when(i > 0)
    def _():
        sum_ref[...] = sum_ref[...] + s1
        sq_ref[...] = sq_ref[...] + s2


def _stats(z, b, D):
    return pl.pallas_call(
        _stats_body,
        grid=(GRID,),
        in_specs=[
            pl.BlockSpec((RB, D), lambda i: (i, 0)),
            pl.BlockSpec((1, D), lambda i: (0, 0)),
        ],
        out_specs=[
            pl.BlockSpec((1, D), lambda i: (0, 0)),
            pl.BlockSpec((1, D), lambda i: (0, 0)),
        ],
        out_shape=[jax.ShapeDtypeStruct((1, D), jnp.float32),
                   jax.ShapeDtypeStruct((1, D), jnp.float32)],
    )(z, b)


def _apply_prep_body(z_ref, b_ref, gw_ref, gb_ref, gms_ref, s1_ref, s2_ref,
                     w_ref, asm_ref, adm_ref, h_ref, as_ref, ad_ref, c_ref,
                     ms_ref, md_ref):
    i = pl.program_id(0)
    t = z_ref[...] + b_ref[...]
    m = s1_ref[...] * (1.0 / N)
    mm = m * gms_ref[...]
    var = s2_ref[...] * (1.0 / N) - 2.0 * mm * m + mm * mm
    y = gw_ref[...] * (t - mm) / jnp.sqrt(var + 1e-5) + gb_ref[...]
    a = jnp.where(y > 0, y, jnp.exp(y) - 1.0)
    h = jnp.dot(a, w_ref[...], preferred_element_type=jnp.float32)
    h_ref[...] = h
    a_s = jnp.dot(h, asm_ref[...], preferred_element_type=jnp.float32)
    a_d = jnp.dot(h, adm_ref[...], preferred_element_type=jnp.float32)
    as_ref[...] = a_s
    ad_ref[...] = a_d
    bs = jnp.max(a_s, axis=0, keepdims=True)
    bd = jnp.max(a_d, axis=0, keepdims=True)

    @pl.when(i == 0)
    def _():
        ms_ref[...] = bs
        md_ref[...] = bd

    @pl.when(i > 0)
    def _():
        ms_ref[...] = jnp.maximum(ms_ref[...], bs)
        md_ref[...] = jnp.maximum(md_ref[...], bd)

    mx = ms_ref[...] + md_ref[...]
    c_ref[...] = jnp.maximum(mx, 0.2 * mx)


def _apply_prep(z, b, gw, gb, gms, s1, s2, w, asm, adm, D, DOUT):
    return pl.pallas_call(
        _apply_prep_body,
        grid=(GRID,),
        in_specs=[
            pl.BlockSpec((RB, D), lambda i: (i, 0)),
            pl.BlockSpec((1, D), lambda i: (0, 0)),
            pl.BlockSpec((1, D), lambda i: (0, 0)),
            pl.BlockSpec((1, D), lambda i: (0, 0)),
            pl.BlockSpec((1, D), lambda i: (0, 0)),
            pl.BlockSpec((1, D), lambda i: (0, 0)),
            pl.BlockSpec((1, D), lambda i: (0, 0)),
            pl.BlockSpec((D, DOUT), lambda i: (0, 0)),
            pl.BlockSpec((DOUT, 16), lambda i: (0, 0)),
            pl.BlockSpec((DOUT, 16), lambda i: (0, 0)),
        ],
        out_specs=[
            pl.BlockSpec((RB, DOUT), lambda i: (i, 0)),
            pl.BlockSpec((RB, 16), lambda i: (i, 0)),
            pl.BlockSpec((RB, 16), lambda i: (i, 0)),
            pl.BlockSpec((1, 16), lambda i: (0, 0)),
        ],
        out_shape=[
            jax.ShapeDtypeStruct((N, DOUT), jnp.float32),
            jax.ShapeDtypeStruct((N, 16), jnp.float32),
            jax.ShapeDtypeStruct((N, 16), jnp.float32),
            jax.ShapeDtypeStruct((1, 16), jnp.float32),
        ],
        scratch_shapes=[pltpu.VMEM((1, 16), jnp.float32),
                        pltpu.VMEM((1, 16), jnp.float32)],
    )(z, b, gw, gb, gms, s1, s2, w, asm, adm)


def _final_body(z_ref, b_ref, o_ref):
    o_ref[...] = z_ref[...] + b_ref[...]


def _final(z, b, D):
    return pl.pallas_call(
        _final_body,
        grid=(GRID,),
        in_specs=[
            pl.BlockSpec((RB, D), lambda i: (i, 0)),
            pl.BlockSpec((1, D), lambda i: (0, 0)),
        ],
        out_specs=pl.BlockSpec((RB, D), lambda i: (i, 0)),
        out_shape=jax.ShapeDtypeStruct((N, D), jnp.float32),
    )(z, b)


def _amap(a, H, F):
    m = jnp.zeros((H * F, 16), jnp.float32)
    for h in range(H):
        m = m.at[h * F:(h + 1) * F, h].set(a[h])
    return m


def kernel(x, edge_index, W1, a1s, a1d, b1, gn1_w, gn1_b, gn1_ms,
           W2, a2s, a2d, b2, gn2_w, gn2_b, gn2_ms, W3, a3s, a3d, b3):
    src = edge_index[0]
    dst = edge_index[1]

    counts = _hist_k(dst)
    off, bptr = _offsets(counts)
    ssrc, sdst = _place_k(src, dst, off)

    xp = jnp.pad(x, ((0, 0), (0, 128 - 11)))
    w1p = jnp.pad(W1, ((0, 128 - 11), (0, 0)))
    h1, as1, ad1, c1 = _prep(xp, w1p, _amap(a1s, 4, 64), _amap(a1d, 4, 64),
                             128, 256)
    z1 = _edge_k1(h1, as1, ad1, ssrc, sdst, bptr, c1)

    s1a, s1b = _stats(z1, b1[None, :], 256)
    h2, as2, ad2, c2 = _apply_prep(z1, b1[None, :], gn1_w[None, :],
                                   gn1_b[None, :], gn1_ms[None, :], s1a, s1b,
                                   W2, _amap(a2s, 4, 96), _amap(a2d, 4, 96),
                                   256, 384)
    z2 = _edge_k2(h2, as2, ad2, ssrc, sdst, bptr, c2)

    s2a, s2b = _stats(z2, b2[None, :], 384)
    h3, as3, ad3, c3 = _apply_prep(z2, b2[None, :], gn2_w[None, :],
                                   gn2_b[None, :], gn2_ms[None, :], s2a, s2b,
                                   W3, _amap(a3s, 1, 48), _amap(a3d, 1, 48),
                                   384, 48)
    z3 = _edge_k3(h3, as3, ad3, ssrc, sdst, bptr, c3)

    return _final(z3, b3[None, :], 48)


# speedup vs baseline: 1.3183x; 1.0354x over previous
import functools
import jax
import jax.numpy as jnp
from jax import lax
from jax.experimental import pallas as pl
from jax.experimental.pallas import tpu as pltpu
from jax.experimental.pallas import tpu_sc as plsc

N = 100000
E = 1600000
NC, NS, L = 2, 16, 16
NW = NC * NS
SPAN = 128
SHIFT = 7
NB = (N + SPAN - 1) // SPAN
NBP = 1024
NP = NB * SPAN
EPW = E // NW
EPAD = 128
HCH = 8192
CH = 512
RB = 1000
GRID = N // RB

_mesh = plsc.VectorSubcoreMesh(core_axis_name="c", subcore_axis_name="s")
_sc_params = pltpu.CompilerParams(use_tc_tiling_on_sc=False)
_sc_params_nl = pltpu.CompilerParams(use_tc_tiling_on_sc=False,
                                     needs_layout_passes=False)

_CHUNKS = []
_o = 0
while _o < EPW:
    _CHUNKS.append((_o, min(HCH, EPW - _o)))
    _o += HCH


def _wid():
    return lax.axis_index("s") * NC + lax.axis_index("c")


def _fill_iota(idx_v, base, count):
    def body(j, _):
        idx_v[pl.ds(j * L, L)] = (jnp.full((L,), base + j * L, jnp.int32)
                                  + lax.iota(jnp.int32, L))
        return 0
    lax.fori_loop(0, count // L, body, 0)


@functools.partial(
    pl.kernel,
    out_type=jax.ShapeDtypeStruct((NW, NBP), jnp.int32),
    mesh=_mesh,
    compiler_params=_sc_params_nl,
    scratch_types=[
        pltpu.VMEM((HCH,), jnp.int32),
        pltpu.VMEM((HCH,), jnp.int32),
        pltpu.VMEM((NBP, L), jnp.int32),
        pltpu.VMEM((NBP,), jnp.int32),
        pltpu.SemaphoreType.DMA,
    ],
)
def _hist_k(dst, cnt_out, idx_v, dst_v, hist2d, cnt_v, sem):
    wid = _wid()
    ebase = wid * EPW
    lane = lax.iota(jnp.int32, L)
    ones = jnp.full((L,), 1, jnp.int32)

    def zh(r, _):
        hist2d[r, :] = jnp.zeros((L,), jnp.int32)
        return 0
    lax.fori_loop(0, NBP, zh, 0)

    for (coff, clen) in _CHUNKS:
        _fill_iota(idx_v, ebase + coff, clen)
        pltpu.async_copy(dst.at[idx_v.at[pl.ds(0, clen)]],
                         dst_v.at[pl.ds(0, clen)], sem).wait()

        def hist(g, _):
            bvec = dst_v[pl.ds(g * L, L)] >> SHIFT
            plsc.addupdate_scatter(hist2d, [bvec, lane], ones)
            return 0
        lax.fori_loop(0, clen // L, hist, 0)

    mask0 = lane == 0

    def red(b, _):
        s = jnp.sum(hist2d[b, :])
        plsc.store_scatter(cnt_v, [jnp.full((L,), b)],
                           jnp.full((L,), s), mask=mask0)
        return 0
    lax.fori_loop(0, NBP, red, 0)
    pltpu.sync_copy(cnt_v, cnt_out.at[wid])


def _off_body(cnt_ref, off_ref, bptr_ref):
    ci = cnt_ref[...]
    tot = jnp.sum(ci, axis=0, keepdims=True)
    s = tot
    k = 1
    while k < NBP:
        s = s + jnp.concatenate(
            [jnp.zeros((1, k), jnp.int32), s[:, :-k]], axis=1)
        k *= 2
    bptr = s - tot
    wp = ci
    k = 1
    while k < NW:
        wp = wp + jnp.concatenate(
            [jnp.zeros((k, NBP), jnp.int32), wp[:-k, :]], axis=0)
        k *= 2
    off_ref[...] = bptr + (wp - ci)
    bptr_ref[...] = bptr


def _offsets(counts):
    return pl.pallas_call(
        _off_body,
        out_shape=[jax.ShapeDtypeStruct((NW, NBP), jnp.int32),
                   jax.ShapeDtypeStruct((1, NBP), jnp.int32)],
    )(counts)


@functools.partial(
    pl.kernel,
    out_type=[jax.ShapeDtypeStruct((E + EPAD,), jnp.int32),
              jax.ShapeDtypeStruct((E + EPAD,), jnp.int32)],
    mesh=_mesh,
    compiler_params=_sc_params_nl,
    scratch_types=[
        pltpu.VMEM((HCH,), jnp.int32),
        pltpu.VMEM((HCH,), jnp.int32),
        pltpu.VMEM((HCH,), jnp.int32),
        pltpu.VMEM((HCH,), jnp.int32),
        pltpu.VMEM((NBP, L), jnp.int32),
        pltpu.VMEM((NBP, L), jnp.int32),
        pltpu.VMEM((NBP,), jnp.int32),
        pltpu.SemaphoreType.DMA,
        pltpu.SemaphoreType.DMA,
    ],
)
def _place_k(src, dst, off, out_src, out_dst, idx_v, src_v, dst_v, pos_v,
             hist2d, off2d, off_v, sem1, sem2):
    wid = _wid()
    ebase = wid * EPW
    lane = lax.iota(jnp.int32, L)
    ones = jnp.full((L,), 1, jnp.int32)

    pltpu.sync_copy(off.at[wid], off_v)

    @pl.when(wid == 0)
    def _():
        _fill_iota(idx_v, E, EPAD)
        for j in range(EPAD // L):
            pos_v[pl.ds(j * L, L)] = jnp.zeros((L,), jnp.int32)
        pltpu.async_copy(pos_v.at[pl.ds(0, EPAD)],
                         out_src.at[idx_v.at[pl.ds(0, EPAD)]], sem1).wait()
        pltpu.async_copy(pos_v.at[pl.ds(0, EPAD)],
                         out_dst.at[idx_v.at[pl.ds(0, EPAD)]], sem2).wait()

    def zh(r, _):
        hist2d[r, :] = jnp.zeros((L,), jnp.int32)
        return 0
    lax.fori_loop(0, NBP, zh, 0)

    for (coff, clen) in _CHUNKS:
        _fill_iota(idx_v, ebase + coff, clen)
        pltpu.async_copy(dst.at[idx_v.at[pl.ds(0, clen)]],
                         dst_v.at[pl.ds(0, clen)], sem1).wait()

        def hist(g, _):
            bvec = dst_v[pl.ds(g * L, L)] >> SHIFT
            plsc.addupdate_scatter(hist2d, [bvec, lane], ones)
            return 0
        lax.fori_loop(0, clen // L, hist, 0)

    def mkoff(b, _):
        row = hist2d[b, :]
        excl = plsc.cumsum(row) - row
        obv = plsc.load_gather(off_v, [jnp.full((L,), b)])
        off2d[b, :] = excl + obv
        return 0
    lax.fori_loop(0, NBP, mkoff, 0)

    for (coff, clen) in _CHUNKS:
        _fill_iota(idx_v, ebase + coff, clen)
        cpa = pltpu.async_copy(src.at[idx_v.at[pl.ds(0, clen)]],
                               src_v.at[pl.ds(0, clen)], sem1)
        cpb = pltpu.async_copy(dst.at[idx_v.at[pl.ds(0, clen)]],
                               dst_v.at[pl.ds(0, clen)], sem2)
        cpa.wait()
        cpb.wait()

        def place(g, _):
            bvec = dst_v[pl.ds(g * L, L)] >> SHIFT
            pos = plsc.load_gather(off2d, [bvec, lane])
            plsc.store_scatter(off2d, [bvec, lane], pos + 1)
            pos_v[pl.ds(g * L, L)] = pos
            return 0
        lax.fori_loop(0, clen // L, place, 0)
        cp1 = pltpu.async_copy(src_v.at[pl.ds(0, clen)],
                               out_src.at[pos_v.at[pl.ds(0, clen)]], sem1)
        cp2 = pltpu.async_copy(dst_v.at[pl.ds(0, clen)],
                               out_dst.at[pos_v.at[pl.ds(0, clen)]], sem2)
        cp1.wait()
        cp2.wait()


def _make_edge_kernel(D, H, F, C):
    nvr = D // L

    @functools.partial(
        pl.kernel,
        out_type=jax.ShapeDtypeStruct((NP, D), jnp.float32),
        mesh=_mesh,
        compiler_params=_sc_params,
        scratch_types=[
            pltpu.VMEM((NBP,), jnp.int32),
            pltpu.VMEM((L,), jnp.float32),
            pltpu.VMEM((C,), jnp.int32),
            pltpu.VMEM((C,), jnp.int32),
            pltpu.VMEM((C,), jnp.int32),
            pltpu.VMEM((C,), jnp.int32),
            pltpu.VMEM((C,), jnp.int32),
            pltpu.VMEM((C,), jnp.int32),
            pltpu.VMEM((C, D), jnp.float32),
            pltpu.VMEM((C, D), jnp.float32),
            pltpu.VMEM((C, L), jnp.float32),
            pltpu.VMEM((C, L), jnp.float32),
            pltpu.VMEM((C, L), jnp.float32),
            pltpu.VMEM((C, L), jnp.float32),
            pltpu.VMEM((SPAN, D), jnp.float32),
            pltpu.VMEM((SPAN, L), jnp.float32),
            pltpu.SemaphoreType.DMA,
            pltpu.SemaphoreType.DMA,
            pltpu.SemaphoreType.DMA,
            pltpu.SemaphoreType.DMA,
            pltpu.SemaphoreType.DMA,
            pltpu.SemaphoreType.DMA,
        ],
    )
    def k(hfeat, asrc, adst, ssrc, sdst, bptr, cvec, out,
          bptr_v, cv, eidx0, eidx1, ssrc0, ssrc1, sdst0, sdst1,
          rows0, rows1, asv0, asv1, adv0, adv1, acc, den,
          si0, si1, sr0, sr1, sa0, sa1):
        eidx = [eidx0, eidx1]
        ssrc_v = [ssrc0, ssrc1]
        sdst_v = [sdst0, sdst1]
        rows = [rows0, rows1]
        asv = [asv0, asv1]
        adv = [adv0, adv1]
        si = [si0, si1]
        sr = [sr0, sr1]
        sa = [sa0, sa1]

        wid = _wid()
        pltpu.sync_copy(bptr.at[0], bptr_v)
        pltpu.sync_copy(cvec.at[0], cv)
        cvv = cv[...]
        nown = (NB - wid + NW - 1) // NW

        def issue_idx(x, e0, ch):
            _fill_iota(eidx[x], e0 + ch * C, C)
            pltpu.async_copy(ssrc.at[eidx[x]], ssrc_v[x], si[x])
            pltpu.async_copy(sdst.at[eidx[x]], sdst_v[x], si[x])

        def wait_idx(x):
            pltpu.make_async_copy(ssrc.at[eidx[x]], ssrc_v[x], si[x]).wait()
            pltpu.make_async_copy(sdst.at[eidx[x]], sdst_v[x], si[x]).wait()

        def issue_rows(x):
            pltpu.async_copy(hfeat.at[ssrc_v[x]], rows[x], sr[x])
            pltpu.async_copy(asrc.at[ssrc_v[x]], asv[x], sa[x])
            pltpu.async_copy(adst.at[sdst_v[x]], adv[x], sa[x])

        def wait_rows(x):
            pltpu.make_async_copy(hfeat.at[ssrc_v[x]], rows[x], sr[x]).wait()
            pltpu.make_async_copy(asrc.at[ssrc_v[x]], asv[x], sa[x]).wait()
            pltpu.make_async_copy(adst.at[sdst_v[x]], adv[x], sa[x]).wait()

        def bucket_body(kk, _):
            b = wid + kk * NW
            nbase = b * SPAN

            def zrow(r, _):
                for j in range(nvr):
                    acc[r, pl.ds(j * L, L)] = jnp.zeros((L,), jnp.float32)
                den[r, :] = jnp.zeros((L,), jnp.float32)
                return 0
            lax.fori_loop(0, SPAN, zrow, 0)

            e0 = bptr_v[pl.ds(b, 1)][0]
            e1 = bptr_v[pl.ds(b + 1, 1)][0]
            ec = e1 - e0
            nch = (ec + C - 1) // C

            @pl.when(nch > 0)
            def _():
                issue_idx(0, e0, 0)
                wait_idx(0)
                issue_rows(0)

                @pl.when(nch > 1)
                def _():
                    issue_idx(1, e0, 1)

            def process(x, ch):
                wait_rows(x)

                @pl.when(ch + 1 < nch)
                def _():
                    wait_idx(1 - x)
                    issue_rows(1 - x)

                rem = jnp.minimum(C, ec - ch * C)
                rws = rows[x]
                sdv = sdst_v[x]
                asx = asv[x]
                adx = adv[x]

                def edge_body(e, _):
                    r = sdv[pl.ds(e, 1)][0] - nbase
                    ev = asx[e, :] + adx[e, :]
                    ev = jnp.maximum(ev, 0.2 * ev)
                    wv = jnp.exp(ev - cvv)
                    plsc.addupdate(den.at[r], wv)
                    for h in range(H):
                        bc = jnp.full((L,), wv[h])
                        for j in range(F // L):
                            o = h * F + j * L
                            plsc.addupdate(acc.at[r, pl.ds(o, L)],
                                           bc * rws[e, pl.ds(o, L)])
                    return 0
                lax.fori_loop(0, rem, edge_body, 0)

                @pl.when(ch + 2 < nch)
                def _():
                    issue_idx(x, e0, ch + 2)

            def chunk_body(ch, _):
                @pl.when(ch % 2 == 0)
                def _():
                    process(0, ch)

                @pl.when(ch % 2 == 1)
                def _():
                    process(1, ch)
                return 0
            lax.fori_loop(0, nch, chunk_body, 0)

            def flush_body(r, _):
                drow = den[r, :]
                for h in range(H):
                    dv = jnp.full((L,), drow[h]) + 1e-16
                    inv = 1.0 / dv
                    for j in range(F // L):
                        o = h * F + j * L
                        acc[r, pl.ds(o, L)] = acc[r, pl.ds(o, L)] * inv
                return 0
            lax.fori_loop(0, SPAN, flush_body, 0)
            pltpu.sync_copy(acc, out.at[pl.ds(nbase, SPAN)])
            return 0
        lax.fori_loop(0, nown, bucket_body, 0)

    return k


_edge_k1 = _make_edge_kernel(256, 4, 64, 64)
_edge_k2 = _make_edge_kernel(384, 4, 96, 64)
_edge_k3 = _make_edge_kernel(48, 1, 48, 128)


def _prep_body(x_ref, w_ref, asm_ref, adm_ref, h_ref, as_ref, ad_ref, c_ref,
               ms_ref, md_ref):
    i = pl.program_id(0)
    h = jnp.dot(x_ref[...], w_ref[...], preferred_element_type=jnp.float32)
    h_ref[...] = h
    a_s = jnp.dot(h, asm_ref[...], preferred_element_type=jnp.float32)
    a_d = jnp.dot(h, adm_ref[...], preferred_element_type=jnp.float32)
    as_ref[...] = a_s
    ad_ref[...] = a_d
    bs = jnp.max(a_s, axis=0, keepdims=True)
    bd = jnp.max(a_d, axis=0, keepdims=True)

    @pl.when(i == 0)
    def _():
        ms_ref[...] = bs
        md_ref[...] = bd

    @pl.when(i > 0)
    def _():
        ms_ref[...] = jnp.maximum(ms_ref[...], bs)
        md_ref[...] = jnp.maximum(md_ref[...], bd)

    m = ms_ref[...] + md_ref[...]
    c_ref[...] = jnp.maximum(m, 0.2 * m)


def _prep(x, w, asm, adm, K, DOUT):
    return pl.pallas_call(
        _prep_body,
        grid=(GRID,),
        in_specs=[
            pl.BlockSpec((RB, K), lambda i: (i, 0)),
            pl.BlockSpec((K, DOUT), lambda i: (0, 0)),
            pl.BlockSpec((DOUT, 16), lambda i: (0, 0)),
            pl.BlockSpec((DOUT, 16), lambda i: (0, 0)),
        ],
        out_specs=[
            pl.BlockSpec((RB, DOUT), lambda i: (i, 0)),
            pl.BlockSpec((RB, 16), lambda i: (i, 0)),
            pl.BlockSpec((RB, 16), lambda i: (i, 0)),
            pl.BlockSpec((1, 16), lambda i: (0, 0)),
        ],
        out_shape=[
            jax.ShapeDtypeStruct((N, DOUT), jnp.float32),
            jax.ShapeDtypeStruct((N, 16), jnp.float32),
            jax.ShapeDtypeStruct((N, 16), jnp.float32),
            jax.ShapeDtypeStruct((1, 16), jnp.float32),
        ],
        scratch_shapes=[pltpu.VMEM((1, 16), jnp.float32),
                        pltpu.VMEM((1, 16), jnp.float32)],
    )(x, w, asm, adm)


def _stats_body(z_ref, b_ref, sum_ref, sq_ref):
    i = pl.program_id(0)
    t = z_ref[...] + b_ref[...]
    s1 = jnp.sum(t, axis=0, keepdims=True)
    s2 = jnp.sum(t * t, axis=0, keepdims=True)

    @pl.when(i == 0)
    def _():
        sum_ref[...] = s1
        sq_ref[...] = s2

    @pl.when(i > 0)
    def _():
        sum_ref[...] = sum_ref[...] + s1
        sq_ref[...] = sq_ref[...] + s2


def _stats(z, b, D):
    return pl.pallas_call(
        _stats_body,
        grid=(GRID,),
        in_specs=[
            pl.BlockSpec((RB, D), lambda i: (i, 0)),
            pl.BlockSpec((1, D), lambda i: (0, 0)),
        ],
        out_specs=[
            pl.BlockSpec((1, D), lambda i: (0, 0)),
            pl.BlockSpec((1, D), lambda i: (0, 0)),
        ],
        out_shape=[jax.ShapeDtypeStruct((1, D), jnp.float32),
                   jax.ShapeDtypeStruct((1, D), jnp.float32)],
    )(z, b)


def _apply_prep_body(z_ref, b_ref, gw_ref, gb_ref, gms_ref, s1_ref, s2_ref,
                     w_ref, asm_ref, adm_ref, h_ref, as_ref, ad_ref, c_ref,
                     ms_ref, md_ref):
    i = pl.program_id(0)
    t = z_ref[...] + b_ref[...]
    m = s1_ref[...] * (1.0 / N)
    mm = m * gms_ref[...]
    var = s2_ref[...] * (1.0 / N) - 2.0 * mm * m + mm * mm
    y = gw_ref[...] * (t - mm) / jnp.sqrt(var + 1e-5) + gb_ref[...]
    a = jnp.where(y > 0, y, jnp.exp(y) - 1.0)
    h = jnp.dot(a, w_ref[...], preferred_element_type=jnp.float32)
    h_ref[...] = h
    a_s = jnp.dot(h, asm_ref[...], preferred_element_type=jnp.float32)
    a_d = jnp.dot(h, adm_ref[...], preferred_element_type=jnp.float32)
    as_ref[...] = a_s
    ad_ref[...] = a_d
    bs = jnp.max(a_s, axis=0, keepdims=True)
    bd = jnp.max(a_d, axis=0, keepdims=True)

    @pl.when(i == 0)
    def _():
        ms_ref[...] = bs
        md_ref[...] = bd

    @pl.when(i > 0)
    def _():
        ms_ref[...] = jnp.maximum(ms_ref[...], bs)
        md_ref[...] = jnp.maximum(md_ref[...], bd)

    mx = ms_ref[...] + md_ref[...]
    c_ref[...] = jnp.maximum(mx, 0.2 * mx)


def _apply_prep(z, b, gw, gb, gms, s1, s2, w, asm, adm, D, DOUT):
    return pl.pallas_call(
        _apply_prep_body,
        grid=(GRID,),
        in_specs=[
            pl.BlockSpec((RB, D), lambda i: (i, 0)),
            pl.BlockSpec((1, D), lambda i: (0, 0)),
            pl.BlockSpec((1, D), lambda i: (0, 0)),
            pl.BlockSpec((1, D), lambda i: (0, 0)),
            pl.BlockSpec((1, D), lambda i: (0, 0)),
            pl.BlockSpec((1, D), lambda i: (0, 0)),
            pl.BlockSpec((1, D), lambda i: (0, 0)),
            pl.BlockSpec((D, DOUT), lambda i: (0, 0)),
            pl.BlockSpec((DOUT, 16), lambda i: (0, 0)),
            pl.BlockSpec((DOUT, 16), lambda i: (0, 0)),
        ],
        out_specs=[
            pl.BlockSpec((RB, DOUT), lambda i: (i, 0)),
            pl.BlockSpec((RB, 16), lambda i: (i, 0)),
            pl.BlockSpec((RB, 16), lambda i: (i, 0)),
            pl.BlockSpec((1, 16), lambda i: (0, 0)),
        ],
        out_shape=[
            jax.ShapeDtypeStruct((N, DOUT), jnp.float32),
            jax.ShapeDtypeStruct((N, 16), jnp.float32),
            jax.ShapeDtypeStruct((N, 16), jnp.float32),
            jax.ShapeDtypeStruct((1, 16), jnp.float32),
        ],
        scratch_shapes=[pltpu.VMEM((1, 16), jnp.float32),
                        pltpu.VMEM((1, 16), jnp.float32)],
    )(z, b, gw, gb, gms, s1, s2, w, asm, adm)


def _final_body(z_ref, b_ref, o_ref):
    o_ref[...] = z_ref[...] + b_ref[...]


def _final(z, b, D):
    return pl.pallas_call(
        _final_body,
        grid=(GRID,),
        in_specs=[
            pl.BlockSpec((RB, D), lambda i: (i, 0)),
            pl.BlockSpec((1, D), lambda i: (0, 0)),
        ],
        out_specs=pl.BlockSpec((RB, D), lambda i: (i, 0)),
        out_shape=jax.ShapeDtypeStruct((N, D), jnp.float32),
    )(z, b)


def _amap(a, H, F):
    m = jnp.zeros((H * F, 16), jnp.float32)
    for h in range(H):
        m = m.at[h * F:(h + 1) * F, h].set(a[h])
    return m


def kernel(x, edge_index, W1, a1s, a1d, b1, gn1_w, gn1_b, gn1_ms,
           W2, a2s, a2d, b2, gn2_w, gn2_b, gn2_ms, W3, a3s, a3d, b3):
    src = edge_index[0]
    dst = edge_index[1]

    counts = _hist_k(dst)
    off, bptr = _offsets(counts)
    ssrc, sdst = _place_k(src, dst, off)

    xp = jnp.pad(x, ((0, 0), (0, 128 - 11)))
    w1p = jnp.pad(W1, ((0, 128 - 11), (0, 0)))
    h1, as1, ad1, c1 = _prep(xp, w1p, _amap(a1s, 4, 64), _amap(a1d, 4, 64),
                             128, 256)
    z1 = _edge_k1(h1, as1, ad1, ssrc, sdst, bptr, c1)

    s1a, s1b = _stats(z1, b1[None, :], 256)
    h2, as2, ad2, c2 = _apply_prep(z1, b1[None, :], gn1_w[None, :],
                                   gn1_b[None, :], gn1_ms[None, :], s1a, s1b,
                                   W2, _amap(a2s, 4, 96), _amap(a2d, 4, 96),
                                   256, 384)
    z2 = _edge_k2(h2, as2, ad2, ssrc, sdst, bptr, c2)

    s2a, s2b = _stats(z2, b2[None, :], 384)
    h3, as3, ad3, c3 = _apply_prep(z2, b2[None, :], gn2_w[None, :],
                                   gn2_b[None, :], gn2_ms[None, :], s2a, s2b,
                                   W3, _amap(a3s, 1, 48), _amap(a3d, 1, 48),
                                   384, 48)
    z3 = _edge_k3(h3, as3, ad3, ssrc, sdst, bptr, c3)

    return _final(z3, b3[None, :], 48)
